# baseline (speedup 1.0000x reference)
import jax
import jax.numpy as jnp
from jax.experimental import pallas as pl


def _dot_nt(a, b):
    return jax.lax.dot_general(
        a, b, (((1,), (1,)), ((), ())), preferred_element_type=jnp.float32)


def _vq_body(x_ref, cb_ref, codes_ref, idx_ref):
    F, N, C = cb_ref.shape
    B = x_ref.shape[0]
    half = jnp.full((8, C), 0.5, jnp.bfloat16)
    iota = jax.lax.broadcasted_iota(jnp.int32, (B, N), 1)
    dists, chs, cls = [], [], []
    for f in range(F):
        cbm = cb_ref[f]
        sq = cbm * cbm
        q1 = sq.astype(jnp.bfloat16)
        r1 = sq - q1.astype(jnp.float32)
        q2 = r1.astype(jnp.bfloat16)
        q3 = (r1 - q2.astype(jnp.float32)).astype(jnp.bfloat16)
        hn = _dot_nt(half, q1) + (_dot_nt(half, q2) + _dot_nt(half, q3))
        ch = cbm.astype(jnp.bfloat16)
        cl = (cbm - ch.astype(jnp.float32)).astype(jnp.bfloat16)
        chs.append(ch)
        cls.append(cl)
        xm = x_ref[:, f * C:(f + 1) * C]
        xh = xm.astype(jnp.bfloat16)
        xl = (xm - xh.astype(jnp.float32)).astype(jnp.bfloat16)
        s = _dot_nt(xh, ch) + (_dot_nt(xh, cl) + _dot_nt(xl, ch))
        dists.append(hn[0:1, :] - s)
    for f in range(F):
        dist = dists[f]
        dmin = jnp.min(dist, axis=1, keepdims=True)
        idx = jnp.min(jnp.where(dist <= dmin, iota, N), axis=1)
        onehot = (iota == idx[:, None]).astype(jnp.bfloat16)
        codes_ref[:, f * C:(f + 1) * C] = (
            jax.lax.dot_general(onehot, chs[f], (((1,), (0,)), ((), ())),
                                preferred_element_type=jnp.float32)
            + jax.lax.dot_general(onehot, cls[f], (((1,), (0,)), ((), ())),
                                  preferred_element_type=jnp.float32))
        idx_ref[f, 0, :] = idx


def kernel(inputs, codebook):
    B, M, C = inputs.shape
    N = codebook.shape[1]
    x2d = inputs.reshape(B, M * C)
    F = 16
    codes2d, idx_m1b = pl.pallas_call(
        _vq_body,
        grid=(M // F,),
        in_specs=[
            pl.BlockSpec((B, F * C), lambda j: (0, j)),
            pl.BlockSpec((F, N, C), lambda j: (j, 0, 0)),
        ],
        out_specs=[
            pl.BlockSpec((B, F * C), lambda j: (0, j)),
            pl.BlockSpec((F, 1, B), lambda j: (j, 0, 0)),
        ],
        out_shape=[
            jax.ShapeDtypeStruct((B, M * C), jnp.float32),
            jax.ShapeDtypeStruct((M, 1, B), jnp.int32),
        ],
    )(x2d, codebook)
    return codes2d.reshape(B, M, C), idx_m1b[:, 0, :].T


# speedup vs baseline: 1.0704x; 1.0704x over previous
import jax
import jax.numpy as jnp
from jax.experimental import pallas as pl


def _dot_nt(a, b):
    return jax.lax.dot_general(
        a, b, (((1,), (1,)), ((), ())), preferred_element_type=jnp.float32)


def _vq_body(x_ref, cb_ref, codes_ref, idx_ref):
    F, N, C = cb_ref.shape
    B = x_ref.shape[0]
    half = jnp.full((8, C), 0.5, jnp.bfloat16)
    iota = jax.lax.broadcasted_iota(jnp.int32, (B, N), 1)
    dists, chs, cls = [], [], []
    for f in range(F):
        cbm = cb_ref[f]
        sq = cbm * cbm
        q1 = sq.astype(jnp.bfloat16)
        r1 = sq - q1.astype(jnp.float32)
        q2 = r1.astype(jnp.bfloat16)
        q3 = (r1 - q2.astype(jnp.float32)).astype(jnp.bfloat16)
        hn = _dot_nt(half, q1) + (_dot_nt(half, q2) + _dot_nt(half, q3))
        ch = cbm.astype(jnp.bfloat16)
        cl = (cbm - ch.astype(jnp.float32)).astype(jnp.bfloat16)
        chs.append(ch)
        cls.append(cl)
        xm = x_ref[:, f * C:(f + 1) * C]
        xh = xm.astype(jnp.bfloat16)
        xl = (xm - xh.astype(jnp.float32)).astype(jnp.bfloat16)
        s = _dot_nt(xh, ch) + (_dot_nt(xh, cl) + _dot_nt(xl, ch))
        dists.append(hn[0:1, :] - s)
    for f in range(F):
        dist = dists[f]
        dmin = jnp.min(dist, axis=1, keepdims=True)
        idx = jnp.min(jnp.where(dist <= dmin, iota, N), axis=1)
        onehot = (iota == idx[:, None]).astype(jnp.bfloat16)
        codes_ref[:, f * C:(f + 1) * C] = (
            jax.lax.dot_general(onehot, chs[f], (((1,), (0,)), ((), ())),
                                preferred_element_type=jnp.float32)
            + jax.lax.dot_general(onehot, cls[f], (((1,), (0,)), ((), ())),
                                  preferred_element_type=jnp.float32))
        idx_ref[f, 0, :] = idx


def kernel(inputs, codebook):
    B, M, C = inputs.shape
    N = codebook.shape[1]
    x2d = inputs.reshape(B, M * C)
    F = 8
    codes2d, idx_m1b = pl.pallas_call(
        _vq_body,
        grid=(M // F,),
        in_specs=[
            pl.BlockSpec((B, F * C), lambda j: (0, j)),
            pl.BlockSpec((F, N, C), lambda j: (j, 0, 0)),
        ],
        out_specs=[
            pl.BlockSpec((B, F * C), lambda j: (0, j)),
            pl.BlockSpec((F, 1, B), lambda j: (j, 0, 0)),
        ],
        out_shape=[
            jax.ShapeDtypeStruct((B, M * C), jnp.float32),
            jax.ShapeDtypeStruct((M, 1, B), jnp.int32),
        ],
    )(x2d, codebook)
    return codes2d.reshape(B, M, C), idx_m1b[:, 0, :].T
